# baseline (speedup 1.0000x reference)
import functools

import jax
import jax.numpy as jnp
from jax import lax
from jax.experimental import pallas as pl
from jax.experimental.pallas import tpu as pltpu
from jax.experimental.pallas import tpu_sc as plsc


@functools.lru_cache(maxsize=None)
def _build_gather(n_tokens: int, hidden: int):
    info = plsc.get_sparse_core_info()
    nc, ns = info.num_cores, info.num_subcores
    nw = nc * ns
    assert n_tokens % nw == 0
    rows_per_w = n_tokens // nw
    chunk = 16
    nbuf = 4
    n_chunks = rows_per_w // chunk

    mesh = plsc.VectorSubcoreMesh(core_axis_name="c", subcore_axis_name="s")

    @functools.partial(
        pl.kernel,
        mesh=mesh,
        out_type=jax.ShapeDtypeStruct((n_tokens, hidden), jnp.float32),
        scratch_types=[
            pltpu.VMEM((rows_per_w,), jnp.int32),
            pltpu.VMEM((nbuf, chunk, hidden), jnp.float32),
        ]
        + [pltpu.SemaphoreType.DMA] * (2 * nbuf),
    )
    def gather_k(table_hbm, idx_hbm, out_hbm, idx_v, bufs, *sems):
        gsems, ssems = sems[:nbuf], sems[nbuf:]
        wid = lax.axis_index("s") * nc + lax.axis_index("c")
        base = wid * rows_per_w
        pltpu.sync_copy(idx_hbm.at[pl.ds(base, rows_per_w)], idx_v)

        def gather_start(i, b):
            pltpu.async_copy(
                table_hbm.at[idx_v.at[pl.ds(i * chunk, chunk)]], bufs.at[b], gsems[b]
            )

        def gather_wait(i, b):
            pltpu.make_async_copy(
                table_hbm.at[idx_v.at[pl.ds(i * chunk, chunk)]], bufs.at[b], gsems[b]
            ).wait()

        def scatter_start(i, b):
            pltpu.async_copy(
                bufs.at[b], out_hbm.at[pl.ds(base + i * chunk, chunk)], ssems[b]
            )

        def scatter_wait(i, b):
            pltpu.make_async_copy(
                bufs.at[b], out_hbm.at[pl.ds(base + i * chunk, chunk)], ssems[b]
            ).wait()

        for b in range(nbuf):
            gather_start(b, b)

        def body(t, carry):
            g = t * nbuf
            for b in range(nbuf):
                gather_wait(g + b, b)
                scatter_start(g + b, b)
            for b in range(nbuf):
                j = g + nbuf + b

                @pl.when(j < n_chunks)
                def _(b=b, j=j):
                    scatter_wait(j - nbuf, b)
                    gather_start(j, b)

            return carry

        lax.fori_loop(0, n_chunks // nbuf, body, 0)

        for b in range(nbuf):
            scatter_wait(n_chunks - nbuf + b, b)

    return gather_k


def kernel(input_ids, embed_weight):
    bsz, seq = input_ids.shape
    vocab, hidden = embed_weight.shape
    ids = input_ids.reshape(-1).astype(jnp.int32)
    flat = _build_gather(bsz * seq, hidden)(embed_weight, ids)
    text_embeds = flat.reshape(bsz, seq, hidden)
    position_ids = jnp.broadcast_to(jnp.arange(seq, dtype=jnp.int32), (bsz, seq))
    lb_loss = jnp.zeros((1,), dtype=text_embeds.dtype)
    return (text_embeds, position_ids, lb_loss)
